# baseline (speedup 1.0000x reference)
import jax
import jax.numpy as jnp
from jax import lax
from jax.experimental import pallas as pl
from jax.experimental.pallas import tpu as pltpu

B, T, H, W, C = 4, 16, 128, 128, 32
NREG = 14
NCT, NCS = 2, 25


def _tc_body(bs_ref, ts_ref, hs_ref, ws_ref, x_ref, tok_ref, out_ref, m_ref):
    ib = pl.program_id(0)
    it = pl.program_id(1)
    out_ref[...] = x_ref[...]

    any_active = jnp.int32(0)
    for i in range(NREG):
        bi = bs_ref[i]
        ti = ts_ref[i]
        t0 = jnp.maximum(ti - NCT, 0)
        t1 = jnp.minimum(ti + NCT, T - 1)
        act = (bi == ib) & (it >= t0) & (it < t1)
        any_active = any_active | act.astype(jnp.int32)
        if i == NREG - 1:
            continue

        hi = hs_ref[i]
        wi = ws_ref[i]
        h0 = jnp.maximum(hi - NCS, 0)
        h1 = jnp.minimum(hi + NCS, H - 1)
        w0 = jnp.maximum(wi - NCS, 0)
        w1 = jnp.minimum(wi + NCS, W - 1)

        @pl.when(act)
        def _(i=i, h0=h0, h1=h1, w0=w0, w1=w1):
            hh = lax.broadcasted_iota(jnp.int32, (H, W, 1), 0)
            ww = lax.broadcasted_iota(jnp.int32, (H, W, 1), 1)
            mask = (hh >= h0) & (hh < h1) & (ww >= w0) & (ww < w1)
            cur = out_ref[0, 0]
            if i < NREG - 2:
                fill = jnp.zeros((1, 1, C), jnp.float32)
            else:
                fill = tok_ref[0, 0][None, :]
            out_ref[0, 0] = jnp.where(mask, fill, cur)

    m_ref[0, 0, 0] = any_active


def _masked_copy(frames, b16, t16, h16, w16, tok):
    out, m = pl.pallas_call(
        _tc_body,
        grid=(B, T),
        in_specs=[
            pl.BlockSpec(memory_space=pltpu.SMEM),
            pl.BlockSpec(memory_space=pltpu.SMEM),
            pl.BlockSpec(memory_space=pltpu.SMEM),
            pl.BlockSpec(memory_space=pltpu.SMEM),
            pl.BlockSpec((1, 1, H, W, C), lambda i, j: (i, j, 0, 0, 0)),
            pl.BlockSpec((1, 1, C), lambda i, j: (0, 0, 0)),
        ],
        out_specs=[
            pl.BlockSpec((1, 1, H, W, C), lambda i, j: (i, j, 0, 0, 0)),
            pl.BlockSpec((1, 1, 1), lambda i, j: (i * T + j, 0, 0),
                         memory_space=pltpu.SMEM),
        ],
        out_shape=[
            jax.ShapeDtypeStruct((B, T, H, W, C), jnp.float32),
            jax.ShapeDtypeStruct((B * T, 1, 1), jnp.int32),
        ],
    )(b16, t16, h16, w16, frames, tok)
    return out, m


def kernel(frames, b, t, h, w, rpos):
    b16 = b[:16].astype(jnp.int32)
    t16 = t[:16].astype(jnp.int32)
    h16 = h[:16].astype(jnp.int32)
    w16 = w[:16].astype(jnp.int32)

    token = frames[rpos[0], rpos[1], rpos[2], rpos[3], :]
    tok = token.reshape(1, 1, C)

    out, m = _masked_copy(frames, b16, t16, h16, w16, tok)
    M = (m[:, 0, 0] != 0).reshape(B, T)
    return out, M


# speedup vs baseline: 3.0014x; 3.0014x over previous
import jax
import jax.numpy as jnp
from jax import lax
from jax.experimental import pallas as pl
from jax.experimental.pallas import tpu as pltpu

B, T, H, W, C = 4, 16, 128, 128, 32
NREG = 14
NCT, NCS = 2, 25


def _tc_body(bs_ref, ts_ref, hs_ref, ws_ref, x_ref, tok_ref, out_ref, m_ref):
    ib = pl.program_id(0)
    it = pl.program_id(1)
    out_ref[...] = x_ref[...]

    any_active = jnp.int32(0)
    for i in range(NREG):
        bi = bs_ref[i]
        ti = ts_ref[i]
        t0 = jnp.maximum(ti - NCT, 0)
        t1 = jnp.minimum(ti + NCT, T - 1)
        act = (bi == ib) & (it >= t0) & (it < t1)
        any_active = any_active | act.astype(jnp.int32)
        if i == NREG - 1:
            continue

        hi = hs_ref[i]
        wi = ws_ref[i]
        h0 = jnp.maximum(hi - NCS, 0)
        h1 = jnp.minimum(hi + NCS, H - 1)
        w0 = jnp.maximum(wi - NCS, 0)
        w1 = jnp.minimum(wi + NCS, W - 1)

        @pl.when(act & (i >= NREG))
        def _(i=i, h0=h0, h1=h1, w0=w0, w1=w1):
            hh = lax.broadcasted_iota(jnp.int32, (H, W, 1), 0)
            ww = lax.broadcasted_iota(jnp.int32, (H, W, 1), 1)
            mask = (hh >= h0) & (hh < h1) & (ww >= w0) & (ww < w1)
            cur = out_ref[0, 0]
            if i < NREG - 2:
                fill = jnp.zeros((1, 1, C), jnp.float32)
            else:
                fill = tok_ref[0, 0][None, :]
            out_ref[0, 0] = jnp.where(mask, fill, cur)

    m_ref[0, 0, 0] = any_active


def _masked_copy(frames, b16, t16, h16, w16, tok):
    out, m = pl.pallas_call(
        _tc_body,
        grid=(B, T),
        in_specs=[
            pl.BlockSpec(memory_space=pltpu.SMEM),
            pl.BlockSpec(memory_space=pltpu.SMEM),
            pl.BlockSpec(memory_space=pltpu.SMEM),
            pl.BlockSpec(memory_space=pltpu.SMEM),
            pl.BlockSpec((1, 1, H, W, C), lambda i, j: (i, j, 0, 0, 0)),
            pl.BlockSpec((1, 1, C), lambda i, j: (0, 0, 0)),
        ],
        out_specs=[
            pl.BlockSpec((1, 1, H, W, C), lambda i, j: (i, j, 0, 0, 0)),
            pl.BlockSpec((1, 1, 1), lambda i, j: (i * T + j, 0, 0),
                         memory_space=pltpu.SMEM),
        ],
        out_shape=[
            jax.ShapeDtypeStruct((B, T, H, W, C), jnp.float32),
            jax.ShapeDtypeStruct((B * T, 1, 1), jnp.int32),
        ],
    )(b16, t16, h16, w16, frames, tok)
    return out, m


def kernel(frames, b, t, h, w, rpos):
    b16 = b[:16].astype(jnp.int32)
    t16 = t[:16].astype(jnp.int32)
    h16 = h[:16].astype(jnp.int32)
    w16 = w[:16].astype(jnp.int32)

    token = frames[rpos[0], rpos[1], rpos[2], rpos[3], :]
    tok = token.reshape(1, 1, C)

    out, m = _masked_copy(frames, b16, t16, h16, w16, tok)
    M = (m[:, 0, 0] != 0).reshape(B, T)
    return out, M


# speedup vs baseline: 8.3738x; 2.7900x over previous
import jax
import jax.numpy as jnp
from jax import lax
from jax.experimental import pallas as pl
from jax.experimental.pallas import tpu as pltpu

B, T, H, W, C = 4, 16, 128, 128, 32
NREG = 14
NCT, NCS = 2, 25


def _tc_body(bs_ref, ts_ref, hs_ref, ws_ref, x_ref, tok_ref, out_ref, m_ref):
    ib = pl.program_id(0)
    it = pl.program_id(1)

    hh = lax.broadcasted_iota(jnp.int32, (H, 1, W), 0)
    ww = lax.broadcasted_iota(jnp.int32, (H, 1, W), 2)

    any_active = jnp.int32(0)
    m1 = jnp.zeros((H, 1, W), jnp.bool_)
    m2 = jnp.zeros((H, 1, W), jnp.bool_)
    for i in range(NREG):
        bi = bs_ref[i]
        ti = ts_ref[i]
        t0 = jnp.maximum(ti - NCT, 0)
        t1 = jnp.minimum(ti + NCT, T - 1)
        act = (bi == ib) & (it >= t0) & (it < t1)
        any_active = any_active | act.astype(jnp.int32)
        if i == NREG - 1:
            continue

        hi = hs_ref[i]
        wi = ws_ref[i]
        h0 = jnp.maximum(hi - NCS, 0)
        h1 = jnp.minimum(hi + NCS, H - 1)
        w0 = jnp.maximum(wi - NCS, 0)
        w1 = jnp.minimum(wi + NCS, W - 1)
        box = ((hh >= h0) & (hh < h1) & (ww >= w0) & (ww < w1)) & act
        if i < NREG - 2:
            m1 = m1 | box
        else:
            m2 = box

    x = x_ref[0, 0]
    tok = tok_ref[0, 0][None]
    out = jnp.where(m2, tok, jnp.where(m1, jnp.float32(0.0), x))
    out_ref[0, 0] = out
    m_ref[0, 0, 0] = any_active


def _masked_copy(frames_t, b16, t16, h16, w16, tok):
    out, m = pl.pallas_call(
        _tc_body,
        grid=(B, T),
        in_specs=[
            pl.BlockSpec(memory_space=pltpu.SMEM),
            pl.BlockSpec(memory_space=pltpu.SMEM),
            pl.BlockSpec(memory_space=pltpu.SMEM),
            pl.BlockSpec(memory_space=pltpu.SMEM),
            pl.BlockSpec((1, 1, H, C, W), lambda i, j: (i, j, 0, 0, 0)),
            pl.BlockSpec((1, 1, C, 1), lambda i, j: (0, 0, 0, 0)),
        ],
        out_specs=[
            pl.BlockSpec((1, 1, H, C, W), lambda i, j: (i, j, 0, 0, 0)),
            pl.BlockSpec((1, 1, 1), lambda i, j: (i * T + j, 0, 0),
                         memory_space=pltpu.SMEM),
        ],
        out_shape=[
            jax.ShapeDtypeStruct((B, T, H, C, W), jnp.float32),
            jax.ShapeDtypeStruct((B * T, 1, 1), jnp.int32),
        ],
    )(b16, t16, h16, w16, frames_t, tok)
    return out, m


def kernel(frames, b, t, h, w, rpos):
    b16 = b[:16].astype(jnp.int32)
    t16 = t[:16].astype(jnp.int32)
    h16 = h[:16].astype(jnp.int32)
    w16 = w[:16].astype(jnp.int32)

    frames_t = jnp.transpose(frames, (0, 1, 2, 4, 3))

    token = frames[rpos[0], rpos[1], rpos[2], rpos[3], :]
    tok = token.reshape(1, 1, C, 1)

    out_t, m = _masked_copy(frames_t, b16, t16, h16, w16, tok)
    out = jnp.transpose(out_t, (0, 1, 2, 4, 3))
    M = (m[:, 0, 0] != 0).reshape(B, T)
    return out, M


# speedup vs baseline: 21.2439x; 2.5370x over previous
import jax
import jax.numpy as jnp
from jax import lax
from jax.experimental import pallas as pl
from jax.experimental.pallas import tpu as pltpu

B, T, H, W, C = 4, 16, 128, 128, 32
NREG = 14
NCT, NCS = 2, 25


def _tc_body(bs_ref, ts_ref, hs_ref, ws_ref, x_ref, tok_ref, out_ref, m_ref):
    ib = pl.program_id(0)
    it = pl.program_id(1)

    hh = lax.broadcasted_iota(jnp.int32, (H, W), 0)
    ww = lax.broadcasted_iota(jnp.int32, (H, W), 1)

    any_active = jnp.int32(0)
    m1 = jnp.zeros((H, W), jnp.bool_)
    m2 = jnp.zeros((H, W), jnp.bool_)
    for i in range(NREG):
        bi = bs_ref[i]
        ti = ts_ref[i]
        t0 = jnp.maximum(ti - NCT, 0)
        t1 = jnp.minimum(ti + NCT, T - 1)
        act = (bi == ib) & (it >= t0) & (it < t1)
        any_active = any_active | act.astype(jnp.int32)
        if i == NREG - 1:
            continue

        hi = hs_ref[i]
        wi = ws_ref[i]
        h0 = jnp.maximum(hi - NCS, 0)
        h1 = jnp.minimum(hi + NCS, H - 1)
        w0 = jnp.maximum(wi - NCS, 0)
        w1 = jnp.minimum(wi + NCS, W - 1)
        box = ((hh >= h0) & (hh < h1) & (ww >= w0) & (ww < w1)) & act
        if i < NREG - 2:
            m1 = m1 | box
        else:
            m2 = box

    x = x_ref[0, 0]
    tok = tok_ref[0, 0][None]
    m1b = m1[:, None, :]
    m2b = m2[:, None, :]
    out = jnp.where(m2b, tok, jnp.where(m1b, jnp.float32(0.0), x))
    out_ref[0, 0] = out
    m_ref[0, 0, 0] = any_active


def _masked_copy(frames_t, b16, t16, h16, w16, tok):
    out, m = pl.pallas_call(
        _tc_body,
        grid=(B, T),
        in_specs=[
            pl.BlockSpec(memory_space=pltpu.SMEM),
            pl.BlockSpec(memory_space=pltpu.SMEM),
            pl.BlockSpec(memory_space=pltpu.SMEM),
            pl.BlockSpec(memory_space=pltpu.SMEM),
            pl.BlockSpec((1, 1, H, C, W), lambda i, j: (i, j, 0, 0, 0)),
            pl.BlockSpec((1, 1, C, 1), lambda i, j: (0, 0, 0, 0)),
        ],
        out_specs=[
            pl.BlockSpec((1, 1, H, C, W), lambda i, j: (i, j, 0, 0, 0)),
            pl.BlockSpec((1, 1, 1), lambda i, j: (i * T + j, 0, 0),
                         memory_space=pltpu.SMEM),
        ],
        out_shape=[
            jax.ShapeDtypeStruct((B, T, H, C, W), jnp.float32),
            jax.ShapeDtypeStruct((B * T, 1, 1), jnp.int32),
        ],
    )(b16, t16, h16, w16, frames_t, tok)
    return out, m


def kernel(frames, b, t, h, w, rpos):
    b16 = b[:16].astype(jnp.int32)
    t16 = t[:16].astype(jnp.int32)
    h16 = h[:16].astype(jnp.int32)
    w16 = w[:16].astype(jnp.int32)

    frames_t = jnp.transpose(frames, (0, 1, 2, 4, 3))

    token = frames[rpos[0], rpos[1], rpos[2], rpos[3], :]
    tok = token.reshape(1, 1, C, 1)

    out_t, m = _masked_copy(frames_t, b16, t16, h16, w16, tok)
    out = jnp.transpose(out_t, (0, 1, 2, 4, 3))
    M = (m[:, 0, 0] != 0).reshape(B, T)
    return out, M
